# baseline (speedup 1.0000x reference)
import jax
import jax.numpy as jnp
from jax.experimental import pallas as pl

C_BLK = 8
N_GENES = 2000
N_EMB = 100
N_BIAS = 128


def _mean_bias_kernel(emb_ref, gix_ref, bias_ref, out_ref):
    x = emb_ref[...]
    s = jnp.sum(x, axis=-1) * (1.0 / N_EMB)
    gix = gix_ref[...]
    bias = bias_ref[...]
    col = jax.lax.broadcasted_iota(jnp.int32, (N_GENES, N_BIAS), 1)
    onehot = gix[0][:, None] == col
    bvals = jnp.sum(jnp.where(onehot, bias, 0.0), axis=1)
    out_ref[...] = s + bvals[None, :]


@jax.jit
def kernel(cell_gene_embedding, gene_ix, bias1):
    n_cells = cell_gene_embedding.shape[0]
    gix2 = gene_ix.astype(jnp.int32).reshape(1, N_GENES)
    bias2 = bias1.reshape(1, N_BIAS)
    grid = (n_cells // C_BLK,)
    return pl.pallas_call(
        _mean_bias_kernel,
        grid=grid,
        in_specs=[
            pl.BlockSpec((C_BLK, N_GENES, N_EMB), lambda i: (i, 0, 0)),
            pl.BlockSpec((1, N_GENES), lambda i: (0, 0)),
            pl.BlockSpec((1, N_BIAS), lambda i: (0, 0)),
        ],
        out_specs=pl.BlockSpec((C_BLK, N_GENES), lambda i: (i, 0)),
        out_shape=jax.ShapeDtypeStruct((n_cells, N_GENES), jnp.float32),
    )(cell_gene_embedding, gix2, bias2)
